# Initial kernel scaffold; baseline (speedup 1.0000x reference)
import jax
import jax.numpy as jnp
from jax.experimental import pallas as pl


def kernel(x, edge_index, W1, b1, W2, b2, Wc, bc):
    raise NotImplementedError("write your pallas kernel here")


# speedup vs baseline: 43.4578x; 43.4578x over previous
import functools

import numpy as np
import jax
import jax.numpy as jnp
from jax import lax
from jax.experimental import pallas as pl
from jax.experimental.pallas import tpu as pltpu
from jax.experimental.pallas import tpu_sc as plsc

N = 10000
F = 6
E = 160000
T = 26
K = 3
H = 7488
OUT = 26
DT = 0.2

N_PAD = 10240
FP = 16
NPT = N_PAD // 16
EPT = E // 16
CW1 = 2000
CH1 = EPT // CW1
CW2 = 400
CH2 = EPT // CW2
WK = 80

_mesh = plsc.VectorSubcoreMesh(core_axis_name="c", subcore_axis_name="s")
_sc_params = pltpu.CompilerParams(use_tc_tiling_on_sc=False)


def _rsqrt_newton(d):
    i = lax.bitcast_convert_type(d, jnp.int32)
    y = lax.bitcast_convert_type(jnp.int32(0x5F3759DF) - (i >> 1), jnp.float32)
    for _ in range(3):
        y = y * (1.5 - 0.5 * d * y * y)
    return y


@functools.partial(
    pl.kernel,
    out_type=(
        jax.ShapeDtypeStruct((T, N_PAD, FP), jnp.float32),
        jax.ShapeDtypeStruct((N_PAD, FP), jnp.float32),
    ),
    mesh=_mesh,
    compiler_params=_sc_params,
    scratch_types=dict(
        src_t=pltpu.VMEM((CH1, 1, CW1), jnp.int32),
        dst_t=pltpu.VMEM((CH1, 1, CW1), jnp.int32),
        gbuf=pltpu.VMEM((CW1, FP), jnp.float32),
        rbuf=pltpu.VMEM((NPT, FP), jnp.float32),
        hbuf=pltpu.VMEM((NPT, FP), jnp.float32),
        ubuf=pltpu.VMEM((NPT, FP), jnp.float32),
        dbuf=pltpu.VMEM((NPT, FP), jnp.float32),
        acc=pltpu.VMEM_SHARED((N_PAD, FP), jnp.float32),
    ),
)
def _euler_sc(src_h, dst_h, xp, ones, zeros, xt, u,
              src_t, dst_t, gbuf, rbuf, hbuf, ubuf, dbuf, acc):
    cid = lax.axis_index("c")
    sid = lax.axis_index("s")

    @pl.when(cid == 0)
    def _():
        nrows = pl.ds(sid * NPT, NPT)

        for c in range(CH1):
            pltpu.sync_copy(src_h.at[pl.ds(sid * EPT + c * CW1, CW1)],
                            src_t.at[c, 0])
            pltpu.sync_copy(dst_h.at[pl.ds(sid * EPT + c * CW1, CW1)],
                            dst_t.at[c, 0])
        pltpu.sync_copy(ones, gbuf)
        pltpu.sync_copy(zeros, acc.at[nrows, :])
        pltpu.sync_copy(xp.at[nrows, :], hbuf)
        plsc.subcore_barrier()

        def deg_chunk(c, _):
            pltpu.sync_copy(gbuf, acc.at[dst_t.at[c, 0]], add=True)
            return 0

        lax.fori_loop(0, CH1, deg_chunk, 0)
        plsc.subcore_barrier()

        pltpu.sync_copy(acc.at[nrows, :], rbuf)
        pltpu.sync_copy(zeros, acc.at[nrows, :])

        def node0(r, _):
            dv = _rsqrt_newton(jnp.maximum(rbuf[r, :], 1.0))
            dbuf[r, :] = dv
            ubuf[r, :] = dv * hbuf[r, :]
            return 0

        lax.fori_loop(0, NPT, node0, 0)
        pltpu.sync_copy(hbuf, xt.at[0].at[nrows, :])
        pltpu.sync_copy(ubuf, u.at[nrows, :])
        plsc.subcore_barrier()

        def euler_iter(t, _):
            def edge_chunk(c, _):
                pltpu.sync_copy(u.at[src_t.at[c, 0]], gbuf)
                pltpu.sync_copy(gbuf, acc.at[dst_t.at[c, 0]], add=True)
                return 0

            lax.fori_loop(0, CH1, edge_chunk, 0)
            plsc.subcore_barrier()
            pltpu.sync_copy(acc.at[nrows, :], rbuf)
            pltpu.sync_copy(zeros, acc.at[nrows, :])

            def node(r, _):
                dv = dbuf[r, :]
                hn = (1.0 - DT) * hbuf[r, :] + DT * (dv * rbuf[r, :])
                hbuf[r, :] = hn
                ubuf[r, :] = dv * hn
                return 0

            lax.fori_loop(0, NPT, node, 0)
            pltpu.sync_copy(hbuf, xt.at[t].at[nrows, :])
            pltpu.sync_copy(ubuf, u.at[nrows, :])
            plsc.subcore_barrier()
            return 0

        lax.fori_loop(1, T, euler_iter, 0)


@functools.partial(
    pl.kernel,
    out_type=jax.ShapeDtypeStruct((K, 2, N_PAD, WK), jnp.float32),
    mesh=_mesh,
    compiler_params=_sc_params,
    scratch_types=dict(
        src_t=pltpu.VMEM((CH2, 1, CW2), jnp.int32),
        dst_t=pltpu.VMEM((CH2, 1, CW2), jnp.int32),
        gbuf=pltpu.VMEM((CW2, WK), jnp.float32),
        acc=pltpu.VMEM_SHARED((N_PAD, WK), jnp.float32),
    ),
)
def _khop_sc(src_h, dst_h, tab, zeros, hk, src_t, dst_t, gbuf, acc):
    cid = lax.axis_index("c")
    sid = lax.axis_index("s")
    nrows = pl.ds(sid * NPT, NPT)

    for c in range(CH2):
        off = sid * EPT + c * CW2
        pltpu.sync_copy(src_h.at[pl.ds(off, CW2)], src_t.at[c, 0])
        pltpu.sync_copy(dst_h.at[pl.ds(off, CW2)], dst_t.at[c, 0])
    pltpu.sync_copy(zeros, acc.at[nrows, :])
    plsc.subcore_barrier()

    for q in range(K):
        srctab = tab.at[cid] if q == 0 else hk.at[q - 1, cid]

        def edge_chunk(c, _):
            pltpu.sync_copy(srctab.at[src_t.at[c, 0]], gbuf)
            pltpu.sync_copy(gbuf, acc.at[dst_t.at[c, 0]], add=True)
            return 0

        lax.fori_loop(0, CH2, edge_chunk, 0)
        plsc.subcore_barrier()
        pltpu.sync_copy(acc.at[nrows, :], hk.at[q, cid].at[nrows, :])
        pltpu.sync_copy(zeros, acc.at[nrows, :])
        plsc.subcore_barrier()


def _moments_body(x_ref, o_ref):
    h = x_ref[...]
    h2 = h * h
    h3 = h2 * h
    h4 = h2 * h2
    for p, hp in enumerate((h, h2, h3, h4)):
        s = jnp.sum(hp, axis=0)
        o_ref[4 * p + 0, :] = s * (1.0 / N)
        o_ref[4 * p + 1, :] = s
        o_ref[4 * p + 2, :] = jnp.max(hp, axis=0)
        o_ref[4 * p + 3, :] = jnp.min(hp, axis=0)


_moments_tc = pl.pallas_call(
    _moments_body,
    out_shape=jax.ShapeDtypeStruct((16, 512), jnp.float32),
)


def _gemv_body(relu, w_ref, v_ref, b_ref, o_ref):
    r = lax.dot_general(v_ref[...], w_ref[0],
                        (((1,), (1,)), ((), ())),
                        preferred_element_type=jnp.float32)
    r = r + b_ref[0]
    if relu:
        r = jnp.maximum(r, 0.0)
    o_ref[0] = r


def _gemv_tc(W, v, b, relu, rows_blk):
    M, Hdim = W.shape
    grid = M // rows_blk
    W3 = W.reshape(grid, rows_blk, Hdim)
    b3 = b.reshape(grid, 1, rows_blk)
    out = pl.pallas_call(
        functools.partial(_gemv_body, relu),
        grid=(grid,),
        in_specs=[
            pl.BlockSpec((1, rows_blk, Hdim), lambda i: (i, 0, 0)),
            pl.BlockSpec((1, Hdim), lambda i: (0, 0)),
            pl.BlockSpec((1, 1, rows_blk), lambda i: (i, 0, 0)),
        ],
        out_specs=pl.BlockSpec((1, 1, rows_blk), lambda i: (i, 0, 0)),
        out_shape=jax.ShapeDtypeStruct((grid, 1, rows_blk), jnp.float32),
    )(W3, v, b3)
    return out.reshape(1, M)


def _build_perm():
    idx = np.zeros((T, K, 4, 4, F), dtype=np.int32)
    for t in range(T):
        for k in range(K):
            for p in range(4):
                for s in range(4):
                    for f in range(F):
                        ch = k * 160 + t * 6 + f
                        idx[t, k, p, s, f] = (4 * p + s) * 512 + ch
    return idx.reshape(-1)


_PERM = _build_perm()


def kernel(x, edge_index, W1, b1, W2, b2, Wc, bc):
    src = edge_index[0]
    dst = edge_index[1]

    xp = jnp.zeros((N_PAD, FP), jnp.float32).at[:N, :F].set(x)
    ones1 = jnp.ones((CW1, FP), jnp.float32)
    zeros1 = jnp.zeros((NPT, FP), jnp.float32)
    zeros2 = jnp.zeros((NPT, WK), jnp.float32)

    xt, _u = _euler_sc(src, dst, xp, ones1, zeros1)

    xt6 = xt[:, :, :F]
    tabf = jnp.transpose(xt6, (1, 0, 2)).reshape(N_PAD, T * F)
    tab = jnp.pad(tabf, ((0, 0), (0, 2 * WK - T * F)))
    tab = tab.reshape(N_PAD, 2, WK).transpose(1, 0, 2)

    hk = _khop_sc(src, dst, tab, zeros2)

    hmat = hk[:, :, :N, :].transpose(2, 0, 1, 3).reshape(N, K * 2 * WK)
    hmat = jnp.pad(hmat, ((0, 0), (0, 512 - K * 2 * WK)))

    mom = _moments_tc(hmat)
    v = jnp.take(mom.reshape(-1), _PERM)

    v1 = _gemv_tc(W1, v.reshape(1, H), b1.reshape(1, H), True, 576)
    v2 = _gemv_tc(W2, v1, b2.reshape(1, H), True, 576)
    Wcp = jnp.zeros((32, H), jnp.float32).at[:OUT].set(Wc)
    bcp = jnp.zeros((1, 32), jnp.float32).at[0, :OUT].set(bc)
    out = _gemv_tc(Wcp, v2, bcp, False, 32)
    return out[0, :OUT]


# speedup vs baseline: 48.8765x; 1.1247x over previous
import functools

import numpy as np
import jax
import jax.numpy as jnp
from jax import lax
from jax.experimental import pallas as pl
from jax.experimental.pallas import tpu as pltpu
from jax.experimental.pallas import tpu_sc as plsc

N = 10000
F = 6
E = 160000
T = 26
K = 3
H = 7488
OUT = 26
DT = 0.2

N_PAD = 10240
FP = 16
NPT = N_PAD // 16
EPT = E // 16
CW1 = 1000
CH1 = EPT // CW1
CW2 = 400
CH2 = EPT // CW2
WK = 104

_mesh = plsc.VectorSubcoreMesh(core_axis_name="c", subcore_axis_name="s")
_sc_params = pltpu.CompilerParams(use_tc_tiling_on_sc=False)


def _rsqrt_newton(d):
    i = lax.bitcast_convert_type(d, jnp.int32)
    y = lax.bitcast_convert_type(jnp.int32(0x5F3759DF) - (i >> 1), jnp.float32)
    for _ in range(3):
        y = y * (1.5 - 0.5 * d * y * y)
    return y


@functools.partial(
    pl.kernel,
    out_type=(
        jax.ShapeDtypeStruct((2, N_PAD, WK), jnp.float32),
        jax.ShapeDtypeStruct((N_PAD, FP), jnp.float32),
    ),
    mesh=_mesh,
    compiler_params=_sc_params,
    scratch_types=dict(
        src_t=pltpu.VMEM((CH1, 1, CW1), jnp.int32),
        dst_t=pltpu.VMEM((CH1, 1, CW1), jnp.int32),
        gbufs=[pltpu.VMEM((CW1, FP), jnp.float32) for _ in range(2)],
        rbuf=pltpu.VMEM((NPT, FP), jnp.float32),
        hbuf=pltpu.VMEM((NPT, FP), jnp.float32),
        ubuf=pltpu.VMEM((NPT, FP), jnp.float32),
        dbuf=pltpu.VMEM((NPT, FP), jnp.float32),
        acc=pltpu.VMEM_SHARED((N_PAD, FP), jnp.float32),
        gsems=[pltpu.SemaphoreType.DMA for _ in range(2)],
    ),
)
def _euler_sc(src_h, dst_h, xp, ones, zeros, tab, u,
              src_t, dst_t, gbufs, rbuf, hbuf, ubuf, dbuf, acc,
              gsems):
    cid = lax.axis_index("c")
    sid = lax.axis_index("s")

    @pl.when(cid == 0)
    def _():
        nrows = pl.ds(sid * NPT, NPT)

        for c in range(CH1):
            pltpu.sync_copy(src_h.at[pl.ds(sid * EPT + c * CW1, CW1)],
                            src_t.at[c, 0])
            pltpu.sync_copy(dst_h.at[pl.ds(sid * EPT + c * CW1, CW1)],
                            dst_t.at[c, 0])
        pltpu.sync_copy(ones, gbufs[0])
        pltpu.sync_copy(zeros, acc.at[nrows, :])
        pltpu.sync_copy(xp.at[nrows, :], hbuf)
        plsc.subcore_barrier()

        def deg_chunk(c, _):
            pltpu.sync_copy(gbufs[0], acc.at[dst_t.at[c, 0]], add=True)
            return 0

        lax.fori_loop(0, CH1, deg_chunk, 0)
        plsc.subcore_barrier()

        pltpu.sync_copy(acc.at[nrows, :], rbuf)
        pltpu.sync_copy(zeros, acc.at[nrows, :])

        def node0(r, _):
            dv = _rsqrt_newton(jnp.maximum(rbuf[r, :], 1.0))
            dbuf[r, :] = dv
            ubuf[r, :] = dv * hbuf[r, :]
            return 0

        lax.fori_loop(0, NPT, node0, 0)
        pltpu.sync_copy(hbuf.at[:, pl.ds(0, 8)],
                        tab.at[0].at[nrows, pl.ds(0, 8)])
        pltpu.sync_copy(ubuf, u.at[nrows, :])
        plsc.subcore_barrier()

        def euler_iter(t, _):
            d0 = pltpu.async_copy(u.at[src_t.at[0, 0]], gbufs[0], gsems[0])
            d1 = pltpu.async_copy(u.at[src_t.at[1, 0]], gbufs[1], gsems[1])
            descs = [d0, d1]
            for c in range(CH1):
                descs[c % 2].wait()
                pltpu.sync_copy(gbufs[c % 2], acc.at[dst_t.at[c, 0]], add=True)
                if c + 2 < CH1:
                    descs[c % 2] = pltpu.async_copy(
                        u.at[src_t.at[c + 2, 0]], gbufs[c % 2], gsems[c % 2])
            plsc.subcore_barrier()
            pltpu.sync_copy(acc.at[nrows, :], rbuf)
            pltpu.sync_copy(zeros, acc.at[nrows, :])

            half = t // 13
            tl = t - half * 13

            def node(r, _):
                dv = dbuf[r, :]
                hn = (1.0 - DT) * hbuf[r, :] + DT * (dv * rbuf[r, :])
                hbuf[r, :] = hn
                ubuf[r, :] = dv * hn
                return 0

            lax.fori_loop(0, NPT, node, 0)
            pltpu.sync_copy(hbuf.at[:, pl.ds(0, 8)],
                            tab.at[half].at[nrows, pl.ds(tl * 8, 8)])
            pltpu.sync_copy(ubuf, u.at[nrows, :])
            plsc.subcore_barrier()
            return 0

        lax.fori_loop(1, T, euler_iter, 0)


@functools.partial(
    pl.kernel,
    out_type=jax.ShapeDtypeStruct((K, 2, N_PAD, WK), jnp.float32),
    mesh=_mesh,
    compiler_params=_sc_params,
    scratch_types=dict(
        src_t=pltpu.VMEM((CH2, 1, CW2), jnp.int32),
        dst_t=pltpu.VMEM((CH2, 1, CW2), jnp.int32),
        gbuf=pltpu.VMEM((CW2, WK), jnp.float32),
        acc=pltpu.VMEM_SHARED((N_PAD, WK), jnp.float32),
    ),
)
def _khop_sc(src_h, dst_h, tab, zeros, hk, src_t, dst_t, gbuf, acc):
    cid = lax.axis_index("c")
    sid = lax.axis_index("s")
    nrows = pl.ds(sid * NPT, NPT)

    for c in range(CH2):
        off = sid * EPT + c * CW2
        pltpu.sync_copy(src_h.at[pl.ds(off, CW2)], src_t.at[c, 0])
        pltpu.sync_copy(dst_h.at[pl.ds(off, CW2)], dst_t.at[c, 0])
    pltpu.sync_copy(zeros, acc.at[nrows, :])
    plsc.subcore_barrier()

    for q in range(K):
        srctab = tab.at[cid] if q == 0 else hk.at[q - 1, cid]

        def edge_chunk(c, _):
            pltpu.sync_copy(srctab.at[src_t.at[c, 0]], gbuf)
            pltpu.sync_copy(gbuf, acc.at[dst_t.at[c, 0]], add=True)
            return 0

        lax.fori_loop(0, CH2, edge_chunk, 0)
        plsc.subcore_barrier()
        pltpu.sync_copy(acc.at[nrows, :], hk.at[q, cid].at[nrows, :])
        pltpu.sync_copy(zeros, acc.at[nrows, :])
        plsc.subcore_barrier()


_MBLK = 1000


def _moments_body(x_ref, o_ref):
    i = pl.program_id(0)
    nblk = pl.num_programs(0)
    h = x_ref[...]
    h2 = h * h
    h3 = h2 * h
    h4 = h2 * h2
    for p, hp in enumerate((h, h2, h3, h4)):
        s = jnp.sum(hp, axis=0)
        mx = jnp.max(hp, axis=0)
        mn = jnp.min(hp, axis=0)

        @pl.when(i == 0)
        def _init():
            o_ref[4 * p + 1, :] = s
            o_ref[4 * p + 2, :] = mx
            o_ref[4 * p + 3, :] = mn

        @pl.when(i > 0)
        def _acc():
            o_ref[4 * p + 1, :] = o_ref[4 * p + 1, :] + s
            o_ref[4 * p + 2, :] = jnp.maximum(o_ref[4 * p + 2, :], mx)
            o_ref[4 * p + 3, :] = jnp.minimum(o_ref[4 * p + 3, :], mn)

        @pl.when(i == nblk - 1)
        def _fin():
            o_ref[4 * p + 0, :] = o_ref[4 * p + 1, :] * (1.0 / N)


_moments_tc = pl.pallas_call(
    _moments_body,
    grid=(N // _MBLK,),
    in_specs=[pl.BlockSpec((_MBLK, 640), lambda i: (i, 0))],
    out_specs=pl.BlockSpec((16, 640), lambda i: (0, 0)),
    out_shape=jax.ShapeDtypeStruct((16, 640), jnp.float32),
)


def _gemv_body(relu, w_ref, v_ref, b_ref, o_ref):
    r = lax.dot_general(v_ref[...], w_ref[0],
                        (((1,), (1,)), ((), ())),
                        preferred_element_type=jnp.float32)
    r = r + b_ref[0]
    if relu:
        r = jnp.maximum(r, 0.0)
    o_ref[0] = r


def _gemv_tc(W, v, b, relu, rows_blk):
    M, Hdim = W.shape
    grid = M // rows_blk
    W3 = W.reshape(grid, rows_blk, Hdim)
    b3 = b.reshape(grid, 1, rows_blk)
    out = pl.pallas_call(
        functools.partial(_gemv_body, relu),
        grid=(grid,),
        in_specs=[
            pl.BlockSpec((1, rows_blk, Hdim), lambda i: (i, 0, 0)),
            pl.BlockSpec((1, Hdim), lambda i: (0, 0)),
            pl.BlockSpec((1, 1, rows_blk), lambda i: (i, 0, 0)),
        ],
        out_specs=pl.BlockSpec((1, 1, rows_blk), lambda i: (i, 0, 0)),
        out_shape=jax.ShapeDtypeStruct((grid, 1, rows_blk), jnp.float32),
    )(W3, v, b3)
    return out.reshape(1, M)


def _build_perm():
    idx = np.zeros((T, K, 4, 4, F), dtype=np.int32)
    for t in range(T):
        for k in range(K):
            for p in range(4):
                for s in range(4):
                    for f in range(F):
                        ch = k * 2 * WK + (t // 13) * WK + (t % 13) * 8 + f
                        idx[t, k, p, s, f] = (4 * p + s) * 640 + ch
    return idx.reshape(-1)


_PERM = _build_perm()


def kernel(x, edge_index, W1, b1, W2, b2, Wc, bc):
    src = edge_index[0]
    dst = edge_index[1]

    xp = jnp.zeros((N_PAD, FP), jnp.float32).at[:N, :F].set(x)
    ones1 = jnp.ones((CW1, FP), jnp.float32)
    zeros1 = jnp.zeros((NPT, FP), jnp.float32)
    zeros2 = jnp.zeros((NPT, WK), jnp.float32)

    tab, _u = _euler_sc(src, dst, xp, ones1, zeros1)

    hk = _khop_sc(src, dst, tab, zeros2)

    hmat = hk[:, :, :N, :].transpose(2, 0, 1, 3).reshape(N, K * 2 * WK)
    hmat = jnp.pad(hmat, ((0, 0), (0, 640 - K * 2 * WK)))

    mom = _moments_tc(hmat)
    v = jnp.take(mom.reshape(-1), _PERM)

    v1 = _gemv_tc(W1, v.reshape(1, H), b1.reshape(1, H), True, 576)
    v2 = _gemv_tc(W2, v1, b2.reshape(1, H), True, 576)
    Wcp = jnp.zeros((32, H), jnp.float32).at[:OUT].set(Wc)
    bcp = jnp.zeros((1, 32), jnp.float32).at[0, :OUT].set(bc)
    out = _gemv_tc(Wcp, v2, bcp, False, 32)
    return out[0, :OUT]


# speedup vs baseline: 56.3078x; 1.1520x over previous
import functools

import numpy as np
import jax
import jax.numpy as jnp
from jax import lax
from jax.experimental import pallas as pl
from jax.experimental.pallas import tpu as pltpu
from jax.experimental.pallas import tpu_sc as plsc

N = 10000
F = 6
E = 160000
T = 26
K = 3
H = 7488
OUT = 26
DT = 0.2

N_PAD = 10240
FP = 16
NPT = N_PAD // 16
EPT = E // 16
CW1 = 1000
CH1 = EPT // CW1
CW2 = 400
CH2 = EPT // CW2
WK = 104

_mesh = plsc.VectorSubcoreMesh(core_axis_name="c", subcore_axis_name="s")
_sc_params = pltpu.CompilerParams(use_tc_tiling_on_sc=False)


def _rsqrt_newton(d):
    i = lax.bitcast_convert_type(d, jnp.int32)
    y = lax.bitcast_convert_type(jnp.int32(0x5F3759DF) - (i >> 1), jnp.float32)
    for _ in range(3):
        y = y * (1.5 - 0.5 * d * y * y)
    return y


@functools.partial(
    pl.kernel,
    out_type=(
        jax.ShapeDtypeStruct((2, N_PAD, WK), jnp.float32),
        jax.ShapeDtypeStruct((N_PAD, FP), jnp.float32),
    ),
    mesh=_mesh,
    compiler_params=_sc_params,
    scratch_types=dict(
        src_t=pltpu.VMEM((CH1, 1, CW1), jnp.int32),
        dst_t=pltpu.VMEM((CH1, 1, CW1), jnp.int32),
        gbufs=[pltpu.VMEM((CW1, FP), jnp.float32) for _ in range(2)],
        rbuf=pltpu.VMEM((NPT, FP), jnp.float32),
        hbuf=pltpu.VMEM((NPT, FP), jnp.float32),
        ubuf=pltpu.VMEM((NPT, FP), jnp.float32),
        dbuf=pltpu.VMEM((NPT, FP), jnp.float32),
        acc=pltpu.VMEM_SHARED((N_PAD, FP), jnp.float32),
        gsems=[pltpu.SemaphoreType.DMA for _ in range(2)],
    ),
)
def _euler_sc(src_h, dst_h, xp, ones, zeros, tab, u,
              src_t, dst_t, gbufs, rbuf, hbuf, ubuf, dbuf, acc,
              gsems):
    cid = lax.axis_index("c")
    sid = lax.axis_index("s")

    @pl.when(cid == 0)
    def _():
        nrows = pl.ds(sid * NPT, NPT)

        for c in range(CH1):
            pltpu.sync_copy(src_h.at[pl.ds(sid * EPT + c * CW1, CW1)],
                            src_t.at[c, 0])
            pltpu.sync_copy(dst_h.at[pl.ds(sid * EPT + c * CW1, CW1)],
                            dst_t.at[c, 0])
        pltpu.sync_copy(ones, gbufs[0])
        pltpu.sync_copy(zeros, acc.at[nrows, :])
        pltpu.sync_copy(xp.at[nrows, :], hbuf)
        plsc.subcore_barrier()

        def deg_chunk(c, _):
            pltpu.sync_copy(gbufs[0], acc.at[dst_t.at[c, 0]], add=True)
            return 0

        lax.fori_loop(0, CH1, deg_chunk, 0)
        plsc.subcore_barrier()

        pltpu.sync_copy(acc.at[nrows, :], rbuf)
        pltpu.sync_copy(zeros, acc.at[nrows, :])

        def node0(r, _):
            dv = _rsqrt_newton(jnp.maximum(rbuf[r, :], 1.0))
            dbuf[r, :] = dv
            ubuf[r, :] = dv * hbuf[r, :]
            return 0

        lax.fori_loop(0, NPT, node0, 0)
        pltpu.sync_copy(hbuf.at[:, pl.ds(0, 8)],
                        tab.at[0].at[nrows, pl.ds(0, 8)])
        pltpu.sync_copy(ubuf, u.at[nrows, :])
        plsc.subcore_barrier()

        def euler_iter(t, _):
            d0 = pltpu.async_copy(u.at[src_t.at[0, 0]], gbufs[0], gsems[0])
            d1 = pltpu.async_copy(u.at[src_t.at[1, 0]], gbufs[1], gsems[1])
            descs = [d0, d1]
            for c in range(CH1):
                descs[c % 2].wait()
                pltpu.sync_copy(gbufs[c % 2], acc.at[dst_t.at[c, 0]], add=True)
                if c + 2 < CH1:
                    descs[c % 2] = pltpu.async_copy(
                        u.at[src_t.at[c + 2, 0]], gbufs[c % 2], gsems[c % 2])
            plsc.subcore_barrier()
            pltpu.sync_copy(acc.at[nrows, :], rbuf)
            pltpu.sync_copy(zeros, acc.at[nrows, :])

            half = t // 13
            tl = t - half * 13

            def node(r, _):
                dv = dbuf[r, :]
                hn = (1.0 - DT) * hbuf[r, :] + DT * (dv * rbuf[r, :])
                hbuf[r, :] = hn
                ubuf[r, :] = dv * hn
                return 0

            lax.fori_loop(0, NPT, node, 0)
            pltpu.sync_copy(hbuf.at[:, pl.ds(0, 8)],
                            tab.at[half].at[nrows, pl.ds(tl * 8, 8)])
            pltpu.sync_copy(ubuf, u.at[nrows, :])
            plsc.subcore_barrier()
            return 0

        lax.fori_loop(1, T, euler_iter, 0)


@functools.partial(
    pl.kernel,
    out_type=(
        jax.ShapeDtypeStruct((N_PAD, 640), jnp.float32),
        jax.ShapeDtypeStruct((2, N_PAD, WK), jnp.float32),
    ),
    mesh=_mesh,
    compiler_params=_sc_params,
    scratch_types=dict(
        src_t=pltpu.VMEM((CH2, 1, CW2), jnp.int32),
        dst_t=pltpu.VMEM((CH2, 1, CW2), jnp.int32),
        gbuf=pltpu.VMEM((CW2, WK), jnp.float32),
        acc=pltpu.VMEM_SHARED((N_PAD, WK), jnp.float32),
    ),
)
def _khop_sc(src_h, dst_h, tab, zeros, hmat, chain, src_t, dst_t, gbuf, acc):
    cid = lax.axis_index("c")
    sid = lax.axis_index("s")
    nrows = pl.ds(sid * NPT, NPT)

    for c in range(CH2):
        off = sid * EPT + c * CW2
        pltpu.sync_copy(src_h.at[pl.ds(off, CW2)], src_t.at[c, 0])
        pltpu.sync_copy(dst_h.at[pl.ds(off, CW2)], dst_t.at[c, 0])
    pltpu.sync_copy(zeros, acc.at[nrows, :])
    plsc.subcore_barrier()

    for q in range(K):
        srctab = tab.at[cid] if q == 0 else chain.at[cid]

        def edge_chunk(c, _):
            pltpu.sync_copy(srctab.at[src_t.at[c, 0]], gbuf)
            pltpu.sync_copy(gbuf, acc.at[dst_t.at[c, 0]], add=True)
            return 0

        lax.fori_loop(0, CH2, edge_chunk, 0)
        plsc.subcore_barrier()
        pltpu.sync_copy(acc.at[nrows, :],
                        hmat.at[nrows, pl.ds(q * 2 * WK + cid * WK, WK)])
        if q < K - 1:
            pltpu.sync_copy(acc.at[nrows, :], chain.at[cid].at[nrows, :])
        pltpu.sync_copy(zeros, acc.at[nrows, :])
        plsc.subcore_barrier()


_MBLK = 1024


def _moments_body(x_ref, o_ref):
    i = pl.program_id(0)
    nblk = pl.num_programs(0)
    h = x_ref[...]
    h2 = h * h
    h3 = h2 * h
    h4 = h2 * h2
    valid = (lax.broadcasted_iota(jnp.int32, (_MBLK, 1), 0)
             + i * _MBLK) < N
    for p, hp in enumerate((h, h2, h3, h4)):
        s = jnp.sum(hp, axis=0)
        mx = jnp.max(jnp.where(valid, hp, -3.0e38), axis=0)
        mn = jnp.min(jnp.where(valid, hp, 3.0e38), axis=0)

        @pl.when(i == 0)
        def _init():
            o_ref[4 * p + 1, :] = s
            o_ref[4 * p + 2, :] = mx
            o_ref[4 * p + 3, :] = mn

        @pl.when(i > 0)
        def _acc():
            o_ref[4 * p + 1, :] = o_ref[4 * p + 1, :] + s
            o_ref[4 * p + 2, :] = jnp.maximum(o_ref[4 * p + 2, :], mx)
            o_ref[4 * p + 3, :] = jnp.minimum(o_ref[4 * p + 3, :], mn)

        @pl.when(i == nblk - 1)
        def _fin():
            o_ref[4 * p + 0, :] = o_ref[4 * p + 1, :] * (1.0 / N)


_moments_tc = pl.pallas_call(
    _moments_body,
    grid=(N_PAD // _MBLK,),
    in_specs=[pl.BlockSpec((_MBLK, 640), lambda i: (i, 0))],
    out_specs=pl.BlockSpec((16, 640), lambda i: (0, 0)),
    out_shape=jax.ShapeDtypeStruct((16, 640), jnp.float32),
)


def _gemv_body(relu, w_ref, v_ref, b_ref, o_ref):
    r = lax.dot_general(v_ref[...], w_ref[0],
                        (((1,), (1,)), ((), ())),
                        preferred_element_type=jnp.float32)
    r = r + b_ref[0]
    if relu:
        r = jnp.maximum(r, 0.0)
    o_ref[0] = r


def _gemv_tc(W, v, b, relu, rows_blk):
    M, Hdim = W.shape
    grid = M // rows_blk
    W3 = W.reshape(grid, rows_blk, Hdim)
    b3 = b.reshape(grid, 1, rows_blk)
    out = pl.pallas_call(
        functools.partial(_gemv_body, relu),
        grid=(grid,),
        in_specs=[
            pl.BlockSpec((1, rows_blk, Hdim), lambda i: (i, 0, 0)),
            pl.BlockSpec((1, Hdim), lambda i: (0, 0)),
            pl.BlockSpec((1, 1, rows_blk), lambda i: (i, 0, 0)),
        ],
        out_specs=pl.BlockSpec((1, 1, rows_blk), lambda i: (i, 0, 0)),
        out_shape=jax.ShapeDtypeStruct((grid, 1, rows_blk), jnp.float32),
    )(W3, v, b3)
    return out.reshape(1, M)


def kernel(x, edge_index, W1, b1, W2, b2, Wc, bc):
    src = edge_index[0]
    dst = edge_index[1]

    xp = jnp.zeros((N_PAD, FP), jnp.float32).at[:N, :F].set(x)
    ones1 = jnp.ones((CW1, FP), jnp.float32)
    zeros1 = jnp.zeros((NPT, FP), jnp.float32)
    zeros2 = jnp.zeros((NPT, WK), jnp.float32)

    tab, _u = _euler_sc(src, dst, xp, ones1, zeros1)

    hmat, _chain = _khop_sc(src, dst, tab, zeros2)

    mom = _moments_tc(hmat)
    m6 = mom[:, :K * 2 * WK].reshape(4, 4, K, 2, 13, 8)[..., :F]
    v = jnp.transpose(m6, (3, 4, 2, 0, 1, 5)).reshape(H)

    v1 = _gemv_tc(W1, v.reshape(1, H), b1.reshape(1, H), True, 576)
    v2 = _gemv_tc(W2, v1, b2.reshape(1, H), True, 576)
    Wcp = jnp.zeros((32, H), jnp.float32).at[:OUT].set(Wc)
    bcp = jnp.zeros((1, 32), jnp.float32).at[0, :OUT].set(bc)
    out = _gemv_tc(Wcp, v2, bcp, False, 32)
    return out[0, :OUT]
